# baseline (speedup 1.0000x reference)
import jax
import jax.numpy as jnp
from jax import lax
from jax.experimental import pallas as pl
from jax.experimental.pallas import tpu as pltpu
from jax.experimental.pallas import tpu_sc as plsc

N = 10000
E = 320000
D = 128
R = 8

NC = 2
NS = 16
NW = NC * NS
CPW = (N * R) // NS

CHC = 125
NCH_C = (E // NW) // CHC
SB = 8

CHE = 125
NCH_E = (E // NW) // CHE


def _sc_counts_body(comb_hbm, zeros_hbm, out0_hbm, out1_hbm, comb_buf,
                    ones_v, counts_sp):
    cid = lax.axis_index("c")
    sid = lax.axis_index("s")
    wid = sid * NC + cid

    FC = (N * R) // 5

    @pl.when(sid < 5)
    def _():
        pltpu.sync_copy(zeros_hbm, counts_sp.at[pl.ds(sid * FC, FC)])

    ov = jnp.ones((16,), jnp.float32)
    for j in range(128 // 16):
        ones_v[pl.ds(j * 16, 16)] = ov
    plsc.subcore_barrier()

    def superchunk(g, _):
        pltpu.sync_copy(comb_hbm.at[wid, pl.ds(g * SB, SB)], comb_buf)

        def body(k, _):
            pltpu.sync_copy(ones_v.at[pl.ds(0, CHC)],
                            counts_sp.at[comb_buf.at[k]], add=True)
            return 0

        lax.fori_loop(0, SB, body, 0)
        return 0

    lax.fori_loop(0, NCH_C // SB, superchunk, 0)
    plsc.subcore_barrier()

    @pl.when(jnp.logical_and(cid == 0, sid < 5))
    def _():
        pltpu.sync_copy(counts_sp.at[pl.ds(sid * FC, FC)],
                        out0_hbm.at[pl.ds(sid * FC, FC)])

    @pl.when(jnp.logical_and(cid == 1, sid < 5))
    def _():
        pltpu.sync_copy(counts_sp.at[pl.ds(sid * FC, FC)],
                        out1_hbm.at[pl.ds(sid * FC, FC)])


NCH_F = (E // NS) // CHC


def _sc_edges_body(t_hbm, lut_hbm, comb2_hbm, gidx_hbm, comb_hbm, dst_hbm,
                   zeros_c_hbm, zeros_hbm, out0_hbm, out1_hbm, comb_cbuf,
                   ones_v, lut_v, gidx_buf, comb_buf, dst_buf, rows_v0,
                   rows_v1, w_v0, w_v1, counts_sp, agg_sp, semr0, semr1,
                   semw0, semw1, sems0, sems1):
    cid = lax.axis_index("c")
    sid = lax.axis_index("s")
    wid = sid * NC + cid

    FC = (N * R) // 5

    pltpu.sync_copy(lut_hbm, lut_v)

    @pl.when(sid < 5)
    def _():
        pltpu.sync_copy(zeros_c_hbm, counts_sp.at[pl.ds(sid * FC, FC)])

    @pl.when(sid < 10)
    def _():
        pltpu.sync_copy(zeros_hbm, agg_sp.at[pl.ds(sid * 1000, 1000)])

    ov = jnp.ones((16,), jnp.float32)
    for j in range(128 // 16):
        ones_v[pl.ds(j * 16, 16)] = ov
    plsc.subcore_barrier()

    def csuper(g, _):
        pltpu.sync_copy(comb2_hbm.at[sid, pl.ds(g * SB, SB)], comb_cbuf)

        def cbody(k, _):
            pltpu.sync_copy(ones_v.at[pl.ds(0, CHC)],
                            counts_sp.at[comb_cbuf.at[k]], add=True)
            return 0

        lax.fori_loop(0, SB, cbody, 0)
        return 0

    lax.fori_loop(0, NCH_F // SB, csuper, 0)
    plsc.subcore_barrier()

    rows = [rows_v0, rows_v1]
    wv = [w_v0, w_v1]
    semr = [semr0, semr1]
    semw = [semw0, semw1]
    sems = [sems0, sems1]

    def superchunk(g, _):
        pltpu.sync_copy(gidx_hbm.at[wid, pl.ds(g * SB, SB)], gidx_buf)
        pltpu.sync_copy(comb_hbm.at[wid, pl.ds(g * SB, SB)], comb_buf)
        pltpu.sync_copy(dst_hbm.at[wid, pl.ds(g * SB, SB)], dst_buf)

        pend_r = pltpu.async_copy(t_hbm.at[gidx_buf.at[0]], rows[0], semr[0])
        pend_w = pltpu.async_copy(counts_sp.at[comb_buf.at[0]], wv[0],
                                  semw[0])
        pend_s = [None, None]
        for k in range(SB):
            b = k % 2
            nb = (k + 1) % 2
            if k + 1 < SB:
                if pend_s[nb] is not None:
                    pend_s[nb].wait()
                    pend_s[nb] = None
                nxt_r = pltpu.async_copy(t_hbm.at[gidx_buf.at[k + 1]],
                                         rows[nb], semr[nb])
                nxt_w = pltpu.async_copy(counts_sp.at[comb_buf.at[k + 1]],
                                         wv[nb], semw[nb])
            pend_w.wait()
            pend_r.wait()
            rb = rows[b]
            wb_v = wv[b]

            def scale(i, _):
                cf = plsc.load_gather(wb_v, [jnp.full((16,), i, jnp.int32)])
                ci = jnp.minimum(cf, 511.0).astype(jnp.int32)
                wb = plsc.load_gather(lut_v, [ci])
                for j in range(D // 16):
                    rb[i, pl.ds(j * 16, 16)] = rb[i, pl.ds(j * 16, 16)] * wb
                return 0

            lax.fori_loop(0, CHE, scale, 0)
            pend_s[b] = pltpu.async_copy(rb, agg_sp.at[dst_buf.at[k]],
                                         sems[b], add=True)
            if k + 1 < SB:
                pend_r, pend_w = nxt_r, nxt_w
        for bb in range(2):
            if pend_s[bb] is not None:
                pend_s[bb].wait()
        return 0

    lax.fori_loop(0, NCH_E // SB, superchunk, 0)
    plsc.subcore_barrier()

    @pl.when(jnp.logical_and(cid == 0, sid < 10))
    def _():
        pltpu.sync_copy(agg_sp.at[pl.ds(sid * 1000, 1000)],
                        out0_hbm.at[pl.ds(sid * 1000, 1000)])

    @pl.when(jnp.logical_and(cid == 1, sid < 10))
    def _():
        pltpu.sync_copy(agg_sp.at[pl.ds(sid * 1000, 1000)],
                        out1_hbm.at[pl.ds(sid * 1000, 1000)])


def _sc_counts(comb2, zeros_c):
    mesh = plsc.VectorSubcoreMesh(core_axis_name="c", subcore_axis_name="s")
    return pl.kernel(
        _sc_counts_body,
        out_type=[jax.ShapeDtypeStruct((N * R,), jnp.float32),
                  jax.ShapeDtypeStruct((N * R,), jnp.float32)],
        mesh=mesh,
        compiler_params=pltpu.CompilerParams(needs_layout_passes=False),
        scratch_types=[
            pltpu.VMEM((SB, CHC), jnp.int32),
            pltpu.VMEM((128,), jnp.float32),
            pltpu.VMEM_SHARED((N * R,), jnp.float32),
        ],
    )(comb2, zeros_c)


def _sc_edges(t, lut, comb2, gidx3, comb3, dst3, zeros_c, zeros_e):
    mesh = plsc.VectorSubcoreMesh(core_axis_name="c", subcore_axis_name="s")
    return pl.kernel(
        _sc_edges_body,
        out_type=[jax.ShapeDtypeStruct((N, D), jnp.float32),
                  jax.ShapeDtypeStruct((N, D), jnp.float32)],
        mesh=mesh,
        compiler_params=pltpu.CompilerParams(needs_layout_passes=False),
        scratch_types=[
            pltpu.VMEM((SB, CHC), jnp.int32),
            pltpu.VMEM((128,), jnp.float32),
            pltpu.VMEM((512,), jnp.float32),
            pltpu.VMEM((SB, CHE), jnp.int32),
            pltpu.VMEM((SB, CHE), jnp.int32),
            pltpu.VMEM((SB, CHE), jnp.int32),
            pltpu.VMEM((CHE, D), jnp.float32),
            pltpu.VMEM((CHE, D), jnp.float32),
            pltpu.VMEM((CHE,), jnp.float32),
            pltpu.VMEM((CHE,), jnp.float32),
            pltpu.VMEM_SHARED((N * R,), jnp.float32),
            pltpu.VMEM_SHARED((N, D), jnp.float32),
            pltpu.SemaphoreType.DMA,
            pltpu.SemaphoreType.DMA,
            pltpu.SemaphoreType.DMA,
            pltpu.SemaphoreType.DMA,
            pltpu.SemaphoreType.DMA,
            pltpu.SemaphoreType.DMA,
        ],
    )(t, lut, comb2, gidx3, comb3, dst3, zeros_c, zeros_e)


NB = 25
BN = N // NB
CB = (N * R) // NB


def _tc_mm_body(x_ref, wrel_ref, wroot_ref, t_ref, xroot_ref):
    r = pl.program_id(1)
    t_ref[...] = jnp.dot(x_ref[...], wrel_ref[0],
                         preferred_element_type=jnp.float32)

    @pl.when(r == 0)
    def _():
        xroot_ref[...] = jnp.dot(x_ref[...], wroot_ref[...],
                                 preferred_element_type=jnp.float32)


def _tc_mm(x, w_rel, w_root):
    return pl.pallas_call(
        _tc_mm_body,
        grid=(NB, R),
        in_specs=[
            pl.BlockSpec((BN, D), lambda i, r: (i, 0)),
            pl.BlockSpec((1, D, D), lambda i, r: (r, 0, 0)),
            pl.BlockSpec((D, D), lambda i, r: (0, 0)),
        ],
        out_specs=[
            pl.BlockSpec((BN, D), lambda i, r: (r * NB + i, 0)),
            pl.BlockSpec((BN, D), lambda i, r: (i, 0)),
        ],
        out_shape=[
            jax.ShapeDtypeStruct((R * N, D), jnp.float32),
            jax.ShapeDtypeStruct((N, D), jnp.float32),
        ],
    )(x, w_rel, w_root)


def _tc_norm_body(counts_ref, norm_ref):
    c = counts_ref[0] + counts_ref[1]
    norm_ref[...] = (1.0 / jnp.maximum(c, 1.0)).reshape(1, 1, CB)


def _tc_norm(counts):
    return pl.pallas_call(
        _tc_norm_body,
        grid=(NB,),
        in_specs=[pl.BlockSpec((NC, CB), lambda i: (0, i))],
        out_specs=pl.BlockSpec((1, 1, CB), lambda i: (i, 0, 0)),
        out_shape=jax.ShapeDtypeStruct((NB, 1, CB), jnp.float32),
    )(counts)


def _tc_post_body(agg0_ref, agg1_ref, xroot_ref, x_ref, b_ref, wau_ref,
                  wax_ref, batt_ref, out_ref):
    u0 = agg0_ref[...] + agg1_ref[...] + xroot_ref[...] + b_ref[...]
    x = x_ref[...]
    s = (jnp.sum(u0 * wau_ref[...], axis=1, keepdims=True)
         + jnp.sum(x * wax_ref[...], axis=1, keepdims=True) + batt_ref[0, 0])
    a1 = jax.nn.sigmoid(s)
    out_ref[...] = jnp.tanh(u0) * a1 + x * (1.0 - a1)


def _tc_post(agg0, agg1, xroot, x, b, wau, wax, batt):
    return pl.pallas_call(
        _tc_post_body,
        grid=(NB,),
        in_specs=[
            pl.BlockSpec((BN, D), lambda i: (i, 0)),
            pl.BlockSpec((BN, D), lambda i: (i, 0)),
            pl.BlockSpec((BN, D), lambda i: (i, 0)),
            pl.BlockSpec((BN, D), lambda i: (i, 0)),
            pl.BlockSpec((1, D), lambda i: (0, 0)),
            pl.BlockSpec((1, D), lambda i: (0, 0)),
            pl.BlockSpec((1, D), lambda i: (0, 0)),
            pl.BlockSpec((1, 1), lambda i: (0, 0)),
        ],
        out_specs=pl.BlockSpec((BN, D), lambda i: (i, 0)),
        out_shape=jax.ShapeDtypeStruct((N, D), jnp.float32),
    )(agg0, agg1, xroot, x, b, wau, wax, batt)


def kernel(node_features, edge_index, edge_type, W_rel, W_root, b, W_att, b_att):
    src = edge_index[0]
    dst = edge_index[1]
    gidx = edge_type * N + src
    comb = dst * R + edge_type

    comb2 = comb.reshape(NS, NCH_F, CHC)
    gidx3 = gidx.reshape(NW, NCH_E, CHE)
    comb3 = comb.reshape(NW, NCH_E, CHE)
    dst3 = dst.reshape(NW, NCH_E, CHE)
    zeros_c = jnp.zeros(((N * R) // 5,), jnp.float32)
    zeros_e = jnp.zeros((1000, D), jnp.float32)
    lut = 1.0 / jnp.maximum(jnp.arange(512, dtype=jnp.float32), 1.0)

    t, xroot = _tc_mm(node_features, W_rel, W_root)
    agg0, agg1 = _sc_edges(t, lut, comb2, gidx3, comb3, dst3, zeros_c,
                           zeros_e)

    wau = W_att[:, :D]
    wax = W_att[:, D:]
    batt = b_att.reshape(1, 1)
    return _tc_post(agg0, agg1, xroot, node_features, b.reshape(1, D), wau,
                    wax, batt)


# speedup vs baseline: 1.2152x; 1.2152x over previous
import jax
import jax.numpy as jnp
from jax import lax
from jax.experimental import pallas as pl
from jax.experimental.pallas import tpu as pltpu
from jax.experimental.pallas import tpu_sc as plsc

N = 10000
E = 320000
D = 128
R = 8

NC = 2
NS = 16
NW = NC * NS
CPW = (N * R) // NS

CHC = 125
NCH_C = (E // NW) // CHC
SB = 8

CHE = 125
NCH_E = (E // NW) // CHE


def _sc_counts_body(comb_hbm, zeros_hbm, out0_hbm, out1_hbm, comb_buf,
                    ones_v, counts_sp):
    cid = lax.axis_index("c")
    sid = lax.axis_index("s")
    wid = sid * NC + cid

    FC = (N * R) // 5

    @pl.when(sid < 5)
    def _():
        pltpu.sync_copy(zeros_hbm, counts_sp.at[pl.ds(sid * FC, FC)])

    ov = jnp.ones((16,), jnp.float32)
    for j in range(128 // 16):
        ones_v[pl.ds(j * 16, 16)] = ov
    plsc.subcore_barrier()

    def superchunk(g, _):
        pltpu.sync_copy(comb_hbm.at[wid, pl.ds(g * SB, SB)], comb_buf)

        def body(k, _):
            pltpu.sync_copy(ones_v.at[pl.ds(0, CHC)],
                            counts_sp.at[comb_buf.at[k]], add=True)
            return 0

        lax.fori_loop(0, SB, body, 0)
        return 0

    lax.fori_loop(0, NCH_C // SB, superchunk, 0)
    plsc.subcore_barrier()

    @pl.when(jnp.logical_and(cid == 0, sid < 5))
    def _():
        pltpu.sync_copy(counts_sp.at[pl.ds(sid * FC, FC)],
                        out0_hbm.at[pl.ds(sid * FC, FC)])

    @pl.when(jnp.logical_and(cid == 1, sid < 5))
    def _():
        pltpu.sync_copy(counts_sp.at[pl.ds(sid * FC, FC)],
                        out1_hbm.at[pl.ds(sid * FC, FC)])


def _sc_edges_body(t_hbm, norm_hbm, gidx_hbm, comb_hbm, dst_hbm, zeros_hbm,
                   out0_hbm, out1_hbm, gidx_buf, comb_buf, dst_buf, rows_v0,
                   rows_v1, w_v0, w_v1, agg_sp, semr0, semr1, semw0, semw1,
                   sems0, sems1):
    cid = lax.axis_index("c")
    sid = lax.axis_index("s")
    wid = sid * NC + cid

    @pl.when(sid < 10)
    def _():
        pltpu.sync_copy(zeros_hbm, agg_sp.at[pl.ds(sid * 1000, 1000)])

    plsc.subcore_barrier()

    rows = [rows_v0, rows_v1]
    wv = [w_v0, w_v1]
    semr = [semr0, semr1]
    semw = [semw0, semw1]
    sems = [sems0, sems1]

    def superchunk(g, _):
        pltpu.sync_copy(gidx_hbm.at[wid, pl.ds(g * SB, SB)], gidx_buf)
        pltpu.sync_copy(comb_hbm.at[wid, pl.ds(g * SB, SB)], comb_buf)
        pltpu.sync_copy(dst_hbm.at[wid, pl.ds(g * SB, SB)], dst_buf)

        pend_r = pltpu.async_copy(t_hbm.at[gidx_buf.at[0]], rows[0], semr[0])
        pend_w = pltpu.async_copy(norm_hbm.at[comb_buf.at[0]], wv[0], semw[0])
        pend_s = [None, None]
        for k in range(SB):
            b = k % 2
            nb = (k + 1) % 2
            if k + 1 < SB:
                if pend_s[nb] is not None:
                    pend_s[nb].wait()
                    pend_s[nb] = None
                nxt_r = pltpu.async_copy(t_hbm.at[gidx_buf.at[k + 1]],
                                         rows[nb], semr[nb])
                nxt_w = pltpu.async_copy(norm_hbm.at[comb_buf.at[k + 1]],
                                         wv[nb], semw[nb])
            pend_w.wait()
            pend_r.wait()
            rb = rows[b]
            wb_v = wv[b]

            def scale(i, _):
                wb = plsc.load_gather(wb_v, [jnp.full((16,), i, jnp.int32)])
                for j in range(D // 16):
                    rb[i, pl.ds(j * 16, 16)] = rb[i, pl.ds(j * 16, 16)] * wb
                return 0

            lax.fori_loop(0, CHE, scale, 0)
            pend_s[b] = pltpu.async_copy(rb, agg_sp.at[dst_buf.at[k]],
                                         sems[b], add=True)
            if k + 1 < SB:
                pend_r, pend_w = nxt_r, nxt_w
        for bb in range(2):
            if pend_s[bb] is not None:
                pend_s[bb].wait()
        return 0

    lax.fori_loop(0, NCH_E // SB, superchunk, 0)
    plsc.subcore_barrier()

    @pl.when(jnp.logical_and(cid == 0, sid < 10))
    def _():
        pltpu.sync_copy(agg_sp.at[pl.ds(sid * 1000, 1000)],
                        out0_hbm.at[pl.ds(sid * 1000, 1000)])

    @pl.when(jnp.logical_and(cid == 1, sid < 10))
    def _():
        pltpu.sync_copy(agg_sp.at[pl.ds(sid * 1000, 1000)],
                        out1_hbm.at[pl.ds(sid * 1000, 1000)])


def _sc_counts(comb2, zeros_c):
    mesh = plsc.VectorSubcoreMesh(core_axis_name="c", subcore_axis_name="s")
    return pl.kernel(
        _sc_counts_body,
        out_type=[jax.ShapeDtypeStruct((N * R,), jnp.float32),
                  jax.ShapeDtypeStruct((N * R,), jnp.float32)],
        mesh=mesh,
        compiler_params=pltpu.CompilerParams(needs_layout_passes=False),
        scratch_types=[
            pltpu.VMEM((SB, CHC), jnp.int32),
            pltpu.VMEM((128,), jnp.float32),
            pltpu.VMEM_SHARED((N * R,), jnp.float32),
        ],
    )(comb2, zeros_c)


def _sc_edges(t, norm, gidx3, comb3, dst3, zeros_e):
    mesh = plsc.VectorSubcoreMesh(core_axis_name="c", subcore_axis_name="s")
    return pl.kernel(
        _sc_edges_body,
        out_type=[jax.ShapeDtypeStruct((N, D), jnp.float32),
                  jax.ShapeDtypeStruct((N, D), jnp.float32)],
        mesh=mesh,
        compiler_params=pltpu.CompilerParams(needs_layout_passes=False),
        scratch_types=[
            pltpu.VMEM((SB, CHE), jnp.int32),
            pltpu.VMEM((SB, CHE), jnp.int32),
            pltpu.VMEM((SB, CHE), jnp.int32),
            pltpu.VMEM((CHE, D), jnp.float32),
            pltpu.VMEM((CHE, D), jnp.float32),
            pltpu.VMEM((CHE,), jnp.float32),
            pltpu.VMEM((CHE,), jnp.float32),
            pltpu.VMEM_SHARED((N, D), jnp.float32),
            pltpu.SemaphoreType.DMA,
            pltpu.SemaphoreType.DMA,
            pltpu.SemaphoreType.DMA,
            pltpu.SemaphoreType.DMA,
            pltpu.SemaphoreType.DMA,
            pltpu.SemaphoreType.DMA,
        ],
    )(t, norm, gidx3, comb3, dst3, zeros_e)


NB = 25
BN = N // NB
CB = (N * R) // NB


def _tc_pre_body(x_ref, wrel_ref, wroot_ref, counts_ref, t_ref, xroot_ref,
                 norm_ref):
    r = pl.program_id(1)
    t_ref[...] = jnp.dot(x_ref[...], wrel_ref[0],
                         preferred_element_type=jnp.float32)

    @pl.when(r == 0)
    def _():
        xroot_ref[...] = jnp.dot(x_ref[...], wroot_ref[...],
                                 preferred_element_type=jnp.float32)
        c = counts_ref[0] + counts_ref[1]
        norm_ref[...] = (1.0 / jnp.maximum(c, 1.0)).reshape(1, 1, CB)


def _tc_pre(x, w_rel, w_root, counts):
    return pl.pallas_call(
        _tc_pre_body,
        grid=(NB, R),
        in_specs=[
            pl.BlockSpec((BN, D), lambda i, r: (i, 0)),
            pl.BlockSpec((1, D, D), lambda i, r: (r, 0, 0)),
            pl.BlockSpec((D, D), lambda i, r: (0, 0)),
            pl.BlockSpec((NC, CB), lambda i, r: (0, i)),
        ],
        out_specs=[
            pl.BlockSpec((BN, D), lambda i, r: (r * NB + i, 0)),
            pl.BlockSpec((BN, D), lambda i, r: (i, 0)),
            pl.BlockSpec((1, 1, CB), lambda i, r: (i, 0, 0)),
        ],
        out_shape=[
            jax.ShapeDtypeStruct((R * N, D), jnp.float32),
            jax.ShapeDtypeStruct((N, D), jnp.float32),
            jax.ShapeDtypeStruct((NB, 1, CB), jnp.float32),
        ],
    )(x, w_rel, w_root, counts)


def _tc_post_body(agg0_ref, agg1_ref, xroot_ref, x_ref, b_ref, wau_ref,
                  wax_ref, batt_ref, out_ref):
    u0 = agg0_ref[...] + agg1_ref[...] + xroot_ref[...] + b_ref[...]
    x = x_ref[...]
    s = (jnp.sum(u0 * wau_ref[...], axis=1, keepdims=True)
         + jnp.sum(x * wax_ref[...], axis=1, keepdims=True) + batt_ref[0, 0])
    a1 = jax.nn.sigmoid(s)
    out_ref[...] = jnp.tanh(u0) * a1 + x * (1.0 - a1)


def _tc_post(agg0, agg1, xroot, x, b, wau, wax, batt):
    return pl.pallas_call(
        _tc_post_body,
        grid=(NB,),
        in_specs=[
            pl.BlockSpec((BN, D), lambda i: (i, 0)),
            pl.BlockSpec((BN, D), lambda i: (i, 0)),
            pl.BlockSpec((BN, D), lambda i: (i, 0)),
            pl.BlockSpec((BN, D), lambda i: (i, 0)),
            pl.BlockSpec((1, D), lambda i: (0, 0)),
            pl.BlockSpec((1, D), lambda i: (0, 0)),
            pl.BlockSpec((1, D), lambda i: (0, 0)),
            pl.BlockSpec((1, 1), lambda i: (0, 0)),
        ],
        out_specs=pl.BlockSpec((BN, D), lambda i: (i, 0)),
        out_shape=jax.ShapeDtypeStruct((N, D), jnp.float32),
    )(agg0, agg1, xroot, x, b, wau, wax, batt)


def kernel(node_features, edge_index, edge_type, W_rel, W_root, b, W_att, b_att):
    src = edge_index[0]
    dst = edge_index[1]
    gidx = edge_type * N + src
    comb = dst * R + edge_type

    comb2 = comb.reshape(NW, NCH_C, CHC)
    gidx3 = gidx.reshape(NW, NCH_E, CHE)
    comb3 = comb.reshape(NW, NCH_E, CHE)
    dst3 = dst.reshape(NW, NCH_E, CHE)
    zeros_c = jnp.zeros(((N * R) // 5,), jnp.float32)
    zeros_e = jnp.zeros((1000, D), jnp.float32)

    c0, c1 = _sc_counts(comb2, zeros_c)
    counts = jnp.stack([c0, c1])
    t, xroot, norm2 = _tc_pre(node_features, W_rel, W_root, counts)
    norm = norm2.reshape(N * R)
    agg0, agg1 = _sc_edges(t, norm, gidx3, comb3, dst3, zeros_e)

    wau = W_att[:, :D]
    wax = W_att[:, D:]
    batt = b_att.reshape(1, 1)
    return _tc_post(agg0, agg1, xroot, node_features, b.reshape(1, D), wau,
                    wax, batt)
